# baseline (speedup 1.0000x reference)
import functools

import jax
import jax.numpy as jnp
from jax import lax
from jax.experimental import pallas as pl
from jax.experimental.pallas import tpu as pltpu
from jax.experimental.pallas import tpu_sc as plsc

N = 10000
D = 128
T = 4
L = 2
E = 320000
NHE = 20000
NNZ = 320000

NC = 2
NS = 16
HD = D // NC

BW = 80
EPT = E // NS
NB = EPT // BW
NGR = BW // 16

RT = 624
MRT = 1248

_mesh = plsc.VectorSubcoreMesh(core_axis_name="c", subcore_axis_name="s")

_f32 = jnp.float32
_i32 = jnp.int32


def _zero16():
    return jnp.zeros((16,), _f32)


HN = 10240
HB = 20480
HTOT = HN + HN + HB
HOWN = HTOT // NS


def _hist_body(col_h, ew_h, node_h, he_h, hp, hstage,
               colb, ewb, nodeb, heb, hl, hl2, tmp, red):
    c = lax.axis_index("c")
    s = lax.axis_index("s")

    ones = jnp.ones((16,), _f32)
    off_dd = jnp.full((16,), HN, _i32)

    def per_t(t, _):
        def zl(i, _):
            hl[pl.ds(i * 16, 16)] = _zero16()
            return 0
        lax.fori_loop(0, (2 * HN) // 16, zl, 0)

        def zl2(i, _):
            hl2[pl.ds(i * 16, 16)] = _zero16()
            return 0
        lax.fori_loop(0, HB // 16, zl2, 0)

        w = s * NC + c
        pltpu.sync_copy(col_h.at[t, w], colb)
        pltpu.sync_copy(ew_h.at[t, w], ewb)
        pltpu.sync_copy(node_h.at[t, w], nodeb)
        pltpu.sync_copy(he_h.at[t, w], heb)

        def scat(i, _):
            r = i // NGR
            k = i - r * NGR
            cv = colb[r, pl.ds(k * 16, 16)]
            ev = ewb[r, pl.ds(k * 16, 16)]
            plsc.addupdate_scatter(hl, [cv], ev)
            nv = nodeb[r, pl.ds(k * 16, 16)]
            plsc.addupdate_scatter(hl, [nv + off_dd], ones)
            hv = heb[r, pl.ds(k * 16, 16)]
            plsc.addupdate_scatter(hl2, [hv], ones)
            return 0
        lax.fori_loop(0, (NB // 2) * NGR, scat, 0)

        pltpu.sync_copy(hl, hstage.at[c, s, 0, pl.ds(0, 2 * HN)])
        pltpu.sync_copy(hl2, hstage.at[c, s, 0, pl.ds(2 * HN, HB)])
        plsc.subcore_barrier()

        pltpu.sync_copy(hstage.at[c, 0, 0, pl.ds(s * HOWN, HOWN)], red)
        for slab in range(1, NS):
            pltpu.sync_copy(hstage.at[c, slab, 0, pl.ds(s * HOWN, HOWN)], tmp)

            def acc_(i, _):
                red[pl.ds(i * 16, 16)] = (red[pl.ds(i * 16, 16)]
                                          + tmp[pl.ds(i * 16, 16)])
                return 0
            lax.fori_loop(0, HOWN // 16, acc_, 0)
        pltpu.sync_copy(red, hp.at[c, t, 0, pl.ds(s * HOWN, HOWN)])
        plsc.subcore_barrier()
        return 0
    lax.fori_loop(0, T, per_t, 0)


_hist = pl.kernel(
    _hist_body,
    out_type=[jax.ShapeDtypeStruct((NC, T, 1, HTOT), _f32),
              jax.ShapeDtypeStruct((NC, NS, 1, HTOT), _f32)],
    mesh=_mesh,
    compiler_params=pltpu.CompilerParams(
        needs_layout_passes=False, use_tc_tiling_on_sc=False),
    scratch_types=[
        pltpu.VMEM((NB // 2, BW), _i32),
        pltpu.VMEM((NB // 2, BW), _f32),
        pltpu.VMEM((NB // 2, BW), _i32),
        pltpu.VMEM((NB // 2, BW), _i32),
        pltpu.VMEM((2 * HN,), _f32),
        pltpu.VMEM((HB,), _f32),
        pltpu.VMEM((HOWN,), _f32),
        pltpu.VMEM((HOWN,), _f32),
    ],
)


NBUF = 5
CH = 50
NCH = NB // CH
MCH = 80


def _scale_rows(rows, nv, k):
    base = k * 16
    for e in range(16):
        b = nv[jnp.full((16,), e, _i32)]
        for q in range(4):
            rows[base + e, pl.ds(q * 16, 16)] = (
                rows[base + e, pl.ds(q * 16, 16)] * b)


def _edges_body(row_h, col_h, ew_h, x2_h, dinv_h,
                node_h, he_h, binv_h, zeros_h,
                sg_out, sh_out, m_hbm,
                idxa, idxb, ewb, gathb, rows, bstage, pool_s, gsem, ssem):
    acc = pool_s
    m_s = pool_s
    c = lax.axis_index("c")
    s = lax.axis_index("s")

    def _zero_acc():
        for u in range(3):
            pltpu.sync_copy(zeros_h, acc.at[pl.ds(s * RT + u * 208, 208)])
        @pl.when(s == NS - 1)
        def _():
            pltpu.sync_copy(zeros_h.at[pl.ds(0, 16)],
                            acc.at[pl.ds(NS * RT, 16)])

    def _flush_acc(dst, t):
        pltpu.sync_copy(acc.at[pl.ds(s * RT, RT)],
                        dst.at[c, t, pl.ds(s * RT, RT)])
        @pl.when(s == NS - 1)
        def _():
            pltpu.sync_copy(acc.at[pl.ds(NS * RT, 16)],
                            dst.at[c, t, pl.ds(NS * RT, 16)])

    x_c = x2_h.at[c]
    m_c = m_hbm.at[c]

    def _gwait(b):
        pltpu.make_async_copy(zeros_h.at[pl.ds(0, BW)], rows.at[b],
                              gsem.at[b]).wait()

    def _swait(b):
        pltpu.make_async_copy(zeros_h.at[pl.ds(0, BW)], rows.at[b],
                              ssem.at[b]).wait()

    def _pipe_pass(x_src, dst, scale, load_idx):
        def step(j, b):
            _gwait(b)
            if scale is not None:
                def grp(k, _):
                    scale(j, b, k)
                    return 0
                lax.fori_loop(0, NGR, grp, 0)
            pltpu.async_copy(rows.at[b], dst.at[idxb.at[j]], ssem.at[b],
                             add=True)

        def chunk(q, _):
            load_idx(q)
            for b in range(NBUF):
                pltpu.async_copy(x_src.at[idxa.at[b]], rows.at[b],
                                 gsem.at[b])

            def blk(p, _):
                for b in range(NBUF):
                    j = p * NBUF + b
                    step(j, b)
                    _swait(b)
                    pltpu.async_copy(x_src.at[idxa.at[j + NBUF]],
                                     rows.at[b], gsem.at[b])
                return 0
            lax.fori_loop(0, CH // NBUF - 1, blk, 0)
            for b in range(NBUF):
                step(CH - NBUF + b, b)
            for b in range(NBUF):
                _swait(b)
            return 0
        lax.fori_loop(0, NCH, chunk, 0)

    def gcn_scale(j, b, k):
        rv = idxa[j, pl.ds(k * 16, 16)]
        cv = idxb[j, pl.ds(k * 16, 16)]
        ev = ewb[j, pl.ds(k * 16, 16)]
        nv = (ev * plsc.load_gather(gathb, [rv])
              * plsc.load_gather(gathb, [cv]))
        _scale_rows(rows.at[b], nv, k)

    def gcn_t(t, _):
        _zero_acc()
        pltpu.sync_copy(dinv_h.at[t, 0], gathb)
        plsc.subcore_barrier()

        def load_gcn(q):
            pltpu.sync_copy(row_h.at[t, s, q], idxa)
            pltpu.sync_copy(col_h.at[t, s, q], idxb)
            pltpu.sync_copy(ew_h.at[t, s, q], ewb)

        _pipe_pass(x_c, acc, gcn_scale, load_gcn)
        plsc.subcore_barrier()

        _flush_acc(sg_out, t)
        plsc.subcore_barrier()
        return 0
    lax.fori_loop(0, T, gcn_t, 0)

    def hyp_t(t, _):
        for u in range(6):
            pltpu.sync_copy(zeros_h, m_s.at[pl.ds(s * MRT + u * 208, 208)])
        @pl.when(s == NS - 1)
        def _():
            pltpu.sync_copy(zeros_h.at[pl.ds(0, 32)],
                            m_s.at[pl.ds(NS * MRT, 32)])
        pltpu.sync_copy(binv_h.at[t, s], bstage)
        plsc.subcore_barrier()

        def load_p1(q):
            pltpu.sync_copy(node_h.at[t, s, q], idxa)
            pltpu.sync_copy(he_h.at[t, s, q], idxb)

        _pipe_pass(x_c, m_s, None, load_p1)
        plsc.subcore_barrier()

        pending = set()
        for i in range(15):
            b = i % NBUF
            if b in pending:
                _swait(b)
            pltpu.sync_copy(m_s.at[pl.ds(s * MRT + i * MCH, MCH)],
                            rows.at[b])
            for k in range(NGR):
                _scale_rows(rows.at[b], bstage[i, pl.ds(k * 16, 16)], k)
            pltpu.async_copy(
                rows.at[b],
                m_c.at[pl.ds(s * MRT + i * MCH, MCH)],
                ssem.at[b])
            pending.add(b)
        b = 15 % NBUF
        if b in pending:
            _swait(b)
            pending.discard(b)
        @pl.when(s == NS - 1)
        def _():
            pltpu.sync_copy(m_s.at[pl.ds(s * MRT + 15 * MCH, 80)],
                            rows.at[b].at[pl.ds(0, 80)])
            for k in range(5):
                _scale_rows(rows.at[b], bstage[15, pl.ds(k * 16, 16)], k)
            pltpu.sync_copy(rows.at[b].at[pl.ds(0, 80)],
                            m_c.at[pl.ds(s * MRT + 15 * MCH, 80)])
        @pl.when(s != NS - 1)
        def _():
            pltpu.sync_copy(m_s.at[pl.ds(s * MRT + 15 * MCH, 48)],
                            rows.at[b].at[pl.ds(0, 48)])
            for k in range(3):
                _scale_rows(rows.at[b], bstage[15, pl.ds(k * 16, 16)], k)
            pltpu.sync_copy(rows.at[b].at[pl.ds(0, 48)],
                            m_c.at[pl.ds(s * MRT + 15 * MCH, 48)])
        for b in pending:
            _swait(b)
        plsc.subcore_barrier()

        _zero_acc()
        plsc.subcore_barrier()

        def load_p2(q):
            pltpu.sync_copy(he_h.at[t, s, q], idxa)
            pltpu.sync_copy(node_h.at[t, s, q], idxb)

        _pipe_pass(m_c, acc, None, load_p2)
        plsc.subcore_barrier()

        _flush_acc(sh_out, t)
        plsc.subcore_barrier()
        return 0
    lax.fori_loop(0, T, hyp_t, 0)


_edges = pl.kernel(
    _edges_body,
    out_type=[
        jax.ShapeDtypeStruct((NC, T, N, HD), _f32),
        jax.ShapeDtypeStruct((NC, T, N, HD), _f32),
        jax.ShapeDtypeStruct((NC, NHE, HD), _f32),
    ],
    mesh=_mesh,
    compiler_params=pltpu.CompilerParams(
        needs_layout_passes=False, use_tc_tiling_on_sc=False),
    scratch_types=[
        pltpu.VMEM((CH, BW), _i32),
        pltpu.VMEM((CH, BW), _i32),
        pltpu.VMEM((CH, BW), _f32),
        pltpu.VMEM((N,), _f32),
        pltpu.VMEM((NBUF, BW, HD), _f32),
        pltpu.VMEM((16, MCH), _f32),
        pltpu.VMEM_SHARED((NHE, HD), _f32),
        pltpu.SemaphoreType.DMA((NBUF,)),
        pltpu.SemaphoreType.DMA((NBUF,)),
    ],
)


NBLK = 400
NGRID = N // NBLK


def _l2n(h):
    n = jnp.sqrt(jnp.sum(h * h, axis=-1, keepdims=True))
    return h / jnp.maximum(n, 1e-12)


def _dense_body(x_ref, sg_ref, sh_ref, aux_ref, wg_ref, bg_ref, wh_ref,
                bh_ref, wi1_ref, wh1_ref, bi1_ref, bh1_ref, wi2_ref, wh2_ref,
                bi2_ref, bh2_ref, hs1_ref, hs2_ref, x1_ref, x2_ref):
    x = x_ref[...]
    G = []
    H = []
    for t in range(T):
        d2 = aux_ref[:, t:t + 1]
        Dv = aux_ref[:, T + t:T + t + 1]
        agg_g = sg_ref[t] + d2 * x
        agg_h = Dv * sh_ref[t]
        gsum = jnp.zeros_like(x)
        hsum = jnp.zeros_like(x)
        for l in range(L):
            hg = jnp.dot(agg_g, wg_ref[t, l],
                         preferred_element_type=_f32,
                         precision=lax.Precision.HIGHEST) + bg_ref[t, l]
            gsum = gsum + _l2n(hg)
            hh = jnp.dot(agg_h, wh_ref[t, l],
                         preferred_element_type=_f32,
                         precision=lax.Precision.HIGHEST) + bh_ref[t, l]
            hsum = hsum + _l2n(hh)
        G.append(gsum)
        H.append(hsum)

    def gru(seq, wiT, whT, bi, bh, hs_ref, xo_ref):
        h = jnp.zeros_like(x)
        for t in range(T):
            gi = jnp.dot(seq[t], wiT, preferred_element_type=_f32) + bi
            gh = jnp.dot(h, whT, preferred_element_type=_f32) + bh
            r = jax.nn.sigmoid(gi[:, :D] + gh[:, :D])
            z = jax.nn.sigmoid(gi[:, D:2 * D] + gh[:, D:2 * D])
            n = jnp.tanh(gi[:, 2 * D:] + r * gh[:, 2 * D:])
            h = (1.0 - z) * n + z * h
            hs_ref[t] = h
        xo_ref[...] = h

    gru(G, wi1_ref[...], wh1_ref[...], bi1_ref[...], bh1_ref[...],
        hs1_ref, x1_ref)
    gru(H, wi2_ref[...], wh2_ref[...], bi2_ref[...], bh2_ref[...],
        hs2_ref, x2_ref)


def _dense(x, S_g, S_h, aux, wg, bg, wh, bh, wi1T, wh1T, bi1, bh1,
           wi2T, wh2T, bi2, bh2):
    blk = lambda i: (i, 0)
    tb = lambda i: (0, i, 0)
    cst2 = lambda i: (0, 0)
    cst4 = lambda i: (0, 0, 0, 0)
    return pl.pallas_call(
        _dense_body,
        grid=(NGRID,),
        in_specs=[
            pl.BlockSpec((NBLK, D), blk),
            pl.BlockSpec((T, NBLK, D), tb),
            pl.BlockSpec((T, NBLK, D), tb),
            pl.BlockSpec((NBLK, D), blk),
            pl.BlockSpec((T, L, D, D), cst4),
            pl.BlockSpec((T, L, 1, D), cst4),
            pl.BlockSpec((T, L, D, D), cst4),
            pl.BlockSpec((T, L, 1, D), cst4),
            pl.BlockSpec((D, 3 * D), cst2),
            pl.BlockSpec((D, 3 * D), cst2),
            pl.BlockSpec((1, 3 * D), cst2),
            pl.BlockSpec((1, 3 * D), cst2),
            pl.BlockSpec((D, 3 * D), cst2),
            pl.BlockSpec((D, 3 * D), cst2),
            pl.BlockSpec((1, 3 * D), cst2),
            pl.BlockSpec((1, 3 * D), cst2),
        ],
        out_specs=[
            pl.BlockSpec((T, NBLK, D), tb),
            pl.BlockSpec((T, NBLK, D), tb),
            pl.BlockSpec((NBLK, D), blk),
            pl.BlockSpec((NBLK, D), blk),
        ],
        out_shape=[
            jax.ShapeDtypeStruct((T, N, D), _f32),
            jax.ShapeDtypeStruct((T, N, D), _f32),
            jax.ShapeDtypeStruct((N, D), _f32),
            jax.ShapeDtypeStruct((N, D), _f32),
        ],
        compiler_params=pltpu.CompilerParams(
            dimension_semantics=("arbitrary",)),
    )(x, S_g, S_h, aux, wg, bg, wh, bh, wi1T, wh1T, bi1, bh1,
      wi2T, wh2T, bi2, bh2)


def kernel(embedding, glo_edge_index, glo_edge_weight, hy_edge_index,
           W_gcn, b_gcn, W_hyp, b_hyp, w_ih1, w_hh1, b_ih1, b_hh1,
           w_ih2, w_hh2, b_ih2, b_hh2):
    X = embedding
    row = glo_edge_index[:, 0, :]
    col = glo_edge_index[:, 1, :]
    node = hy_edge_index[:, 0, :]
    he = hy_edge_index[:, 1, :]

    col_h = col.reshape(T, NS, NCH, CH, BW)
    ew_h = glo_edge_weight.reshape(T, NS, NCH, CH, BW)
    node_h = node.reshape(T, NS, NCH, CH, BW)
    he_h = he.reshape(T, NS, NCH, CH, BW)
    col_h32 = col.reshape(T, 2 * NS, NB // 2, BW)
    ew_h32 = glo_edge_weight.reshape(T, 2 * NS, NB // 2, BW)
    node_h32 = node.reshape(T, 2 * NS, NB // 2, BW)
    he_h32 = he.reshape(T, 2 * NS, NB // 2, BW)
    row_h = row.reshape(T, NS, NCH, CH, BW)
    x2 = jnp.stack([X[:, :HD], X[:, HD:]], axis=0)

    hp, _hs = _hist(col_h32, ew_h32, node_h32, he_h32)
    hsum = hp[0, :, 0] + hp[1, :, 0]
    deg = hsum[:, :N] + 1.0
    dinv = 1.0 / jnp.sqrt(deg)
    dd = hsum[:, HN:HN + N]
    Dinv = 1.0 / jnp.maximum(dd, 1.0)
    bd = hsum[:, 2 * HN:2 * HN + NHE]
    binv = 1.0 / jnp.maximum(bd, 1.0)
    dinv_in = dinv[:, None, :]
    bidx = (jnp.arange(NS) * MRT)[:, None] + jnp.arange(16 * MCH)[None, :]
    binv_tiles = binv[:, bidx].reshape(T, NS, 16, MCH)
    zeros_in = jnp.zeros((208, HD), _f32)
    aux = jnp.zeros((N, D), _f32)
    aux = aux.at[:, 0:T].set((dinv * dinv).T).at[:, T:2 * T].set(Dinv.T)

    sg, sh, _m = _edges(row_h, col_h, ew_h, x2, dinv_in,
                        node_h, he_h, binv_tiles, zeros_in)
    S_g = jnp.concatenate([sg[0], sg[1]], axis=-1)
    S_h = jnp.concatenate([sh[0], sh[1]], axis=-1)

    hs1, hs2, x1, x2o = _dense(
        X, S_g, S_h, aux, W_gcn, b_gcn.reshape(T, L, 1, D), W_hyp,
        b_hyp.reshape(T, L, 1, D), w_ih1.T, w_hh1.T, b_ih1.reshape(1, 3 * D),
        b_hh1.reshape(1, 3 * D), w_ih2.T, w_hh2.T, b_ih2.reshape(1, 3 * D),
        b_hh2.reshape(1, 3 * D))
    h1 = jnp.transpose(hs1, (1, 0, 2))
    h2 = jnp.transpose(hs2, (1, 0, 2))
    return (x1, x2o, h1, h2)


# speedup vs baseline: 1.2335x; 1.2335x over previous
import functools

import jax
import jax.numpy as jnp
from jax import lax
from jax.experimental import pallas as pl
from jax.experimental.pallas import tpu as pltpu
from jax.experimental.pallas import tpu_sc as plsc

N = 10000
D = 128
T = 4
L = 2
E = 320000
NHE = 20000
NNZ = 320000

NC = 2
NS = 16
HD = D // NC

BW = 80
EPT = E // NS
NB = EPT // BW
NGR = BW // 16

RT = 624
MRT = 1248

_mesh = plsc.VectorSubcoreMesh(core_axis_name="c", subcore_axis_name="s")

_f32 = jnp.float32
_i32 = jnp.int32


def _zero16():
    return jnp.zeros((16,), _f32)


HN = 10240
HB = 20480
HTOT = HN + HN + HB
HOWN = HTOT // NS


def _hist_body(col_h, ew_h, node_h, he_h, hp, hstage,
               colb, ewb, nodeb, heb, hl, hl2, tmp, red):
    c = lax.axis_index("c")
    s = lax.axis_index("s")

    ones = jnp.ones((16,), _f32)
    off_dd = jnp.full((16,), HN, _i32)

    def per_t(t, _):
        def zl(i, _):
            hl[pl.ds(i * 16, 16)] = _zero16()
            return 0
        lax.fori_loop(0, (2 * HN) // 16, zl, 0)

        def zl2(i, _):
            hl2[pl.ds(i * 16, 16)] = _zero16()
            return 0
        lax.fori_loop(0, HB // 16, zl2, 0)

        w = s * NC + c
        pltpu.sync_copy(col_h.at[t, w], colb)
        pltpu.sync_copy(ew_h.at[t, w], ewb)
        pltpu.sync_copy(node_h.at[t, w], nodeb)
        pltpu.sync_copy(he_h.at[t, w], heb)

        def scat(i, _):
            r = i // NGR
            k = i - r * NGR
            cv = colb[r, pl.ds(k * 16, 16)]
            ev = ewb[r, pl.ds(k * 16, 16)]
            plsc.addupdate_scatter(hl, [cv], ev)
            nv = nodeb[r, pl.ds(k * 16, 16)]
            plsc.addupdate_scatter(hl, [nv + off_dd], ones)
            hv = heb[r, pl.ds(k * 16, 16)]
            plsc.addupdate_scatter(hl2, [hv], ones)
            return 0
        lax.fori_loop(0, (NB // 2) * NGR, scat, 0)

        pltpu.sync_copy(hl, hstage.at[c, s, 0, pl.ds(0, 2 * HN)])
        pltpu.sync_copy(hl2, hstage.at[c, s, 0, pl.ds(2 * HN, HB)])
        plsc.subcore_barrier()

        pltpu.sync_copy(hstage.at[c, 0, 0, pl.ds(s * HOWN, HOWN)], red)
        for slab in range(1, NS):
            pltpu.sync_copy(hstage.at[c, slab, 0, pl.ds(s * HOWN, HOWN)], tmp)

            def acc_(i, _):
                red[pl.ds(i * 16, 16)] = (red[pl.ds(i * 16, 16)]
                                          + tmp[pl.ds(i * 16, 16)])
                return 0
            lax.fori_loop(0, HOWN // 16, acc_, 0)
        pltpu.sync_copy(red, hp.at[c, t, 0, pl.ds(s * HOWN, HOWN)])
        plsc.subcore_barrier()
        return 0
    lax.fori_loop(0, T, per_t, 0)


_hist = pl.kernel(
    _hist_body,
    out_type=[jax.ShapeDtypeStruct((NC, T, 1, HTOT), _f32),
              jax.ShapeDtypeStruct((NC, NS, 1, HTOT), _f32)],
    mesh=_mesh,
    compiler_params=pltpu.CompilerParams(
        needs_layout_passes=False, use_tc_tiling_on_sc=False),
    scratch_types=[
        pltpu.VMEM((NB // 2, BW), _i32),
        pltpu.VMEM((NB // 2, BW), _f32),
        pltpu.VMEM((NB // 2, BW), _i32),
        pltpu.VMEM((NB // 2, BW), _i32),
        pltpu.VMEM((2 * HN,), _f32),
        pltpu.VMEM((HB,), _f32),
        pltpu.VMEM((HOWN,), _f32),
        pltpu.VMEM((HOWN,), _f32),
    ],
)


NBUF = 5
CH = 50
NCH = NB // CH
MCH = 80


def _scale_rows(rows, nv, k):
    base = k * 16
    for e in range(16):
        b = nv[jnp.full((16,), e, _i32)]
        for q in range(4):
            rows[base + e, pl.ds(q * 16, 16)] = (
                rows[base + e, pl.ds(q * 16, 16)] * b)


def _edges_body(row_adj_h, col_h, ew_h, x2_h, dinv_h,
                node_adj_h, he_h, node_h, he_adj_h, binv_h, zeros_h,
                sg_out, sh_out, m_hbm,
                idxa, idxb, ewb, gathb, rows, bstage, pool_s, gsem, ssem):
    acc = pool_s
    m_s = pool_s
    c = lax.axis_index("c")
    s = lax.axis_index("s")

    def _zero_acc():
        for u in range(3):
            pltpu.sync_copy(zeros_h, acc.at[pl.ds(s * RT + u * 208, 208)])
        @pl.when(s == NS - 1)
        def _():
            pltpu.sync_copy(zeros_h.at[pl.ds(0, 16)],
                            acc.at[pl.ds(NS * RT, 16)])

    def _flush_acc(dst, t):
        pltpu.sync_copy(acc.at[pl.ds(s * RT, RT)],
                        dst.at[c, t, pl.ds(s * RT, RT)])
        @pl.when(s == NS - 1)
        def _():
            pltpu.sync_copy(acc.at[pl.ds(NS * RT, 16)],
                            dst.at[c, t, pl.ds(NS * RT, 16)])

    def _gwait(b):
        pltpu.make_async_copy(x2_h.at[pl.ds(0, BW)], rows.at[b],
                              gsem.at[b]).wait()

    def _swait(b):
        pltpu.make_async_copy(x2_h.at[pl.ds(0, BW)], rows.at[b],
                              ssem.at[b]).wait()

    def _pipe_pass(x_src, dst, scale, load_idx):
        def step(j, b):
            _gwait(b)
            if scale is not None:
                def grp(k, _):
                    scale(j, b, k)
                    return 0
                lax.fori_loop(0, NGR, grp, 0)
            pltpu.async_copy(rows.at[b], dst.at[idxb.at[j]], ssem.at[b],
                             add=True)

        def chunk(q, _):
            load_idx(q)
            for b in range(NBUF):
                pltpu.async_copy(x_src.at[idxa.at[b]], rows.at[b],
                                 gsem.at[b])

            def blk(p, _):
                for b in range(NBUF):
                    j = p * NBUF + b
                    step(j, b)
                    _swait(b)
                    pltpu.async_copy(x_src.at[idxa.at[j + NBUF]],
                                     rows.at[b], gsem.at[b])
                return 0
            lax.fori_loop(0, CH // NBUF - 1, blk, 0)
            for b in range(NBUF):
                step(CH - NBUF + b, b)
            for b in range(NBUF):
                _swait(b)
            return 0
        lax.fori_loop(0, NCH, chunk, 0)

    def gcn_scale(j, b, k):
        rv = idxa[j, pl.ds(k * 16, 16)] - c * N
        cv = idxb[j, pl.ds(k * 16, 16)]
        ev = ewb[j, pl.ds(k * 16, 16)]
        nv = (ev * plsc.load_gather(gathb, [rv])
              * plsc.load_gather(gathb, [cv]))
        _scale_rows(rows.at[b], nv, k)

    def gcn_t(t, _):
        _zero_acc()
        pltpu.sync_copy(dinv_h.at[t, 0], gathb)
        plsc.subcore_barrier()

        def load_gcn(q):
            pltpu.sync_copy(row_adj_h.at[c, t, s, q], idxa)
            pltpu.sync_copy(col_h.at[t, s, q], idxb)
            pltpu.sync_copy(ew_h.at[t, s, q], ewb)

        _pipe_pass(x2_h, acc, gcn_scale, load_gcn)
        plsc.subcore_barrier()

        _flush_acc(sg_out, t)
        plsc.subcore_barrier()
        return 0
    lax.fori_loop(0, T, gcn_t, 0)

    def hyp_t(t, _):
        for u in range(6):
            pltpu.sync_copy(zeros_h, m_s.at[pl.ds(s * MRT + u * 208, 208)])
        @pl.when(s == NS - 1)
        def _():
            pltpu.sync_copy(zeros_h.at[pl.ds(0, 32)],
                            m_s.at[pl.ds(NS * MRT, 32)])
        pltpu.sync_copy(binv_h.at[t, s], bstage)
        plsc.subcore_barrier()

        def load_p1(q):
            pltpu.sync_copy(node_adj_h.at[c, t, s, q], idxa)
            pltpu.sync_copy(he_h.at[t, s, q], idxb)

        _pipe_pass(x2_h, m_s, None, load_p1)
        plsc.subcore_barrier()

        pending = set()
        for i in range(15):
            b = i % NBUF
            if b in pending:
                _swait(b)
            pltpu.sync_copy(m_s.at[pl.ds(s * MRT + i * MCH, MCH)],
                            rows.at[b])
            for k in range(NGR):
                _scale_rows(rows.at[b], bstage[i, pl.ds(k * 16, 16)], k)
            pltpu.async_copy(
                rows.at[b],
                m_hbm.at[pl.ds(c * NHE + s * MRT + i * MCH, MCH)],
                ssem.at[b])
            pending.add(b)
        b = 15 % NBUF
        if b in pending:
            _swait(b)
            pending.discard(b)
        @pl.when(s == NS - 1)
        def _():
            pltpu.sync_copy(m_s.at[pl.ds(s * MRT + 15 * MCH, 80)],
                            rows.at[b].at[pl.ds(0, 80)])
            for k in range(5):
                _scale_rows(rows.at[b], bstage[15, pl.ds(k * 16, 16)], k)
            pltpu.sync_copy(rows.at[b].at[pl.ds(0, 80)],
                            m_hbm.at[pl.ds(c * NHE + s * MRT + 15 * MCH, 80)])
        @pl.when(s != NS - 1)
        def _():
            pltpu.sync_copy(m_s.at[pl.ds(s * MRT + 15 * MCH, 48)],
                            rows.at[b].at[pl.ds(0, 48)])
            for k in range(3):
                _scale_rows(rows.at[b], bstage[15, pl.ds(k * 16, 16)], k)
            pltpu.sync_copy(rows.at[b].at[pl.ds(0, 48)],
                            m_hbm.at[pl.ds(c * NHE + s * MRT + 15 * MCH, 48)])
        for b in pending:
            _swait(b)
        plsc.subcore_barrier()

        _zero_acc()
        plsc.subcore_barrier()

        def load_p2(q):
            pltpu.sync_copy(he_adj_h.at[c, t, s, q], idxa)
            pltpu.sync_copy(node_h.at[t, s, q], idxb)

        _pipe_pass(m_hbm, acc, None, load_p2)
        plsc.subcore_barrier()

        _flush_acc(sh_out, t)
        plsc.subcore_barrier()
        return 0
    lax.fori_loop(0, T, hyp_t, 0)


_edges = pl.kernel(
    _edges_body,
    out_type=[
        jax.ShapeDtypeStruct((NC, T, N, HD), _f32),
        jax.ShapeDtypeStruct((NC, T, N, HD), _f32),
        jax.ShapeDtypeStruct((NC * NHE, HD), _f32),
    ],
    mesh=_mesh,
    compiler_params=pltpu.CompilerParams(
        needs_layout_passes=False, use_tc_tiling_on_sc=False),
    scratch_types=[
        pltpu.VMEM((CH, BW), _i32),
        pltpu.VMEM((CH, BW), _i32),
        pltpu.VMEM((CH, BW), _f32),
        pltpu.VMEM((N,), _f32),
        pltpu.VMEM((NBUF, BW, HD), _f32),
        pltpu.VMEM((16, MCH), _f32),
        pltpu.VMEM_SHARED((NHE, HD), _f32),
        pltpu.SemaphoreType.DMA((NBUF,)),
        pltpu.SemaphoreType.DMA((NBUF,)),
    ],
)


NBLK = 400
NGRID = N // NBLK


def _l2n(h):
    n = jnp.sqrt(jnp.sum(h * h, axis=-1, keepdims=True))
    return h / jnp.maximum(n, 1e-12)


def _dense_body(x_ref, sg_ref, sh_ref, aux_ref, wg_ref, bg_ref, wh_ref,
                bh_ref, wi1_ref, wh1_ref, bi1_ref, bh1_ref, wi2_ref, wh2_ref,
                bi2_ref, bh2_ref, hs1_ref, hs2_ref, x1_ref, x2_ref):
    x = x_ref[...]
    G = []
    H = []
    for t in range(T):
        d2 = aux_ref[:, t:t + 1]
        Dv = aux_ref[:, T + t:T + t + 1]
        agg_g = sg_ref[t] + d2 * x
        agg_h = Dv * sh_ref[t]
        gsum = jnp.zeros_like(x)
        hsum = jnp.zeros_like(x)
        for l in range(L):
            hg = jnp.dot(agg_g, wg_ref[t, l],
                         preferred_element_type=_f32,
                         precision=lax.Precision.HIGHEST) + bg_ref[t, l]
            gsum = gsum + _l2n(hg)
            hh = jnp.dot(agg_h, wh_ref[t, l],
                         preferred_element_type=_f32,
                         precision=lax.Precision.HIGHEST) + bh_ref[t, l]
            hsum = hsum + _l2n(hh)
        G.append(gsum)
        H.append(hsum)

    def gru(seq, wiT, whT, bi, bh, hs_ref, xo_ref):
        h = jnp.zeros_like(x)
        for t in range(T):
            gi = jnp.dot(seq[t], wiT, preferred_element_type=_f32) + bi
            gh = jnp.dot(h, whT, preferred_element_type=_f32) + bh
            r = jax.nn.sigmoid(gi[:, :D] + gh[:, :D])
            z = jax.nn.sigmoid(gi[:, D:2 * D] + gh[:, D:2 * D])
            n = jnp.tanh(gi[:, 2 * D:] + r * gh[:, 2 * D:])
            h = (1.0 - z) * n + z * h
            hs_ref[t] = h
        xo_ref[...] = h

    gru(G, wi1_ref[...], wh1_ref[...], bi1_ref[...], bh1_ref[...],
        hs1_ref, x1_ref)
    gru(H, wi2_ref[...], wh2_ref[...], bi2_ref[...], bh2_ref[...],
        hs2_ref, x2_ref)


def _dense(x, S_g, S_h, aux, wg, bg, wh, bh, wi1T, wh1T, bi1, bh1,
           wi2T, wh2T, bi2, bh2):
    blk = lambda i: (i, 0)
    tb = lambda i: (0, i, 0)
    cst2 = lambda i: (0, 0)
    cst4 = lambda i: (0, 0, 0, 0)
    return pl.pallas_call(
        _dense_body,
        grid=(NGRID,),
        in_specs=[
            pl.BlockSpec((NBLK, D), blk),
            pl.BlockSpec((T, NBLK, D), tb),
            pl.BlockSpec((T, NBLK, D), tb),
            pl.BlockSpec((NBLK, D), blk),
            pl.BlockSpec((T, L, D, D), cst4),
            pl.BlockSpec((T, L, 1, D), cst4),
            pl.BlockSpec((T, L, D, D), cst4),
            pl.BlockSpec((T, L, 1, D), cst4),
            pl.BlockSpec((D, 3 * D), cst2),
            pl.BlockSpec((D, 3 * D), cst2),
            pl.BlockSpec((1, 3 * D), cst2),
            pl.BlockSpec((1, 3 * D), cst2),
            pl.BlockSpec((D, 3 * D), cst2),
            pl.BlockSpec((D, 3 * D), cst2),
            pl.BlockSpec((1, 3 * D), cst2),
            pl.BlockSpec((1, 3 * D), cst2),
        ],
        out_specs=[
            pl.BlockSpec((T, NBLK, D), tb),
            pl.BlockSpec((T, NBLK, D), tb),
            pl.BlockSpec((NBLK, D), blk),
            pl.BlockSpec((NBLK, D), blk),
        ],
        out_shape=[
            jax.ShapeDtypeStruct((T, N, D), _f32),
            jax.ShapeDtypeStruct((T, N, D), _f32),
            jax.ShapeDtypeStruct((N, D), _f32),
            jax.ShapeDtypeStruct((N, D), _f32),
        ],
        compiler_params=pltpu.CompilerParams(
            dimension_semantics=("arbitrary",)),
    )(x, S_g, S_h, aux, wg, bg, wh, bh, wi1T, wh1T, bi1, bh1,
      wi2T, wh2T, bi2, bh2)


def kernel(embedding, glo_edge_index, glo_edge_weight, hy_edge_index,
           W_gcn, b_gcn, W_hyp, b_hyp, w_ih1, w_hh1, b_ih1, b_hh1,
           w_ih2, w_hh2, b_ih2, b_hh2):
    X = embedding
    row = glo_edge_index[:, 0, :]
    col = glo_edge_index[:, 1, :]
    node = hy_edge_index[:, 0, :]
    he = hy_edge_index[:, 1, :]

    col_h = col.reshape(T, NS, NCH, CH, BW)
    ew_h = glo_edge_weight.reshape(T, NS, NCH, CH, BW)
    node_h = node.reshape(T, NS, NCH, CH, BW)
    he_h = he.reshape(T, NS, NCH, CH, BW)
    col_h32 = col.reshape(T, 2 * NS, NB // 2, BW)
    ew_h32 = glo_edge_weight.reshape(T, 2 * NS, NB // 2, BW)
    node_h32 = node.reshape(T, 2 * NS, NB // 2, BW)
    he_h32 = he.reshape(T, 2 * NS, NB // 2, BW)
    row_adj = jnp.stack([row, row + N], 0).reshape(NC, T, NS, NCH, CH, BW)
    node_adj = jnp.stack([node, node + N], 0).reshape(NC, T, NS, NCH, CH, BW)
    he_adj = jnp.stack([he, he + NHE], 0).reshape(NC, T, NS, NCH, CH, BW)
    x2 = jnp.concatenate([X[:, :HD], X[:, HD:]], axis=0)

    hp, _hs = _hist(col_h32, ew_h32, node_h32, he_h32)
    hsum = hp[0, :, 0] + hp[1, :, 0]
    deg = hsum[:, :N] + 1.0
    dinv = 1.0 / jnp.sqrt(deg)
    dd = hsum[:, HN:HN + N]
    Dinv = 1.0 / jnp.maximum(dd, 1.0)
    bd = hsum[:, 2 * HN:2 * HN + NHE]
    binv = 1.0 / jnp.maximum(bd, 1.0)
    dinv_in = dinv[:, None, :]
    bidx = (jnp.arange(NS) * MRT)[:, None] + jnp.arange(16 * MCH)[None, :]
    binv_tiles = binv[:, bidx].reshape(T, NS, 16, MCH)
    zeros_in = jnp.zeros((208, HD), _f32)
    aux = jnp.zeros((N, D), _f32)
    aux = aux.at[:, 0:T].set((dinv * dinv).T).at[:, T:2 * T].set(Dinv.T)

    sg, sh, _m = _edges(row_adj, col_h, ew_h, x2, dinv_in,
                        node_adj, he_h, node_h, he_adj, binv_tiles, zeros_in)
    S_g = jnp.concatenate([sg[0], sg[1]], axis=-1)
    S_h = jnp.concatenate([sh[0], sh[1]], axis=-1)

    hs1, hs2, x1, x2o = _dense(
        X, S_g, S_h, aux, W_gcn, b_gcn.reshape(T, L, 1, D), W_hyp,
        b_hyp.reshape(T, L, 1, D), w_ih1.T, w_hh1.T, b_ih1.reshape(1, 3 * D),
        b_hh1.reshape(1, 3 * D), w_ih2.T, w_hh2.T, b_ih2.reshape(1, 3 * D),
        b_hh2.reshape(1, 3 * D))
    h1 = jnp.transpose(hs1, (1, 0, 2))
    h2 = jnp.transpose(hs2, (1, 0, 2))
    return (x1, x2o, h1, h2)
